# baseline (speedup 1.0000x reference)
import jax
import jax.numpy as jnp
from jax import lax
from jax.experimental import pallas as pl
from jax.experimental.pallas import tpu as pltpu
from jax.experimental.pallas import tpu_sc as plsc

N = 10000
E = 160000
D = 256
DG = 128
DH = 64
NQ = 4
NC = 2
NS = 16
L = 16

EPS = E // NS
K = 80
NBATCH = EPS // K
GRP = 4
NGRP = 31
ROWS_PER_SUB = N // NS


def _gcn_body(row3, col3, val3, x2, out_hbm,
              row_m, col_m, val_m, g0, g1, lo0, lo1, hi0, hi1,
              acc_lo, acc_hi, sem, sem_s, sem_m):
    c = lax.axis_index("c")
    s = lax.axis_index("s")
    gbufs = (g0, g1)
    lobufs = (lo0, lo1)
    hibufs = (hi0, hi1)

    def meta_start(b, m):
        pltpu.async_copy(row3.at[s, b], row_m.at[m], sem_m)
        pltpu.async_copy(col3.at[s, b], col_m.at[m], sem_m)
        pltpu.async_copy(val3.at[s, b], val_m.at[m], sem_m)

    def meta_wait(b, m):
        pltpu.make_async_copy(row3.at[s, b], row_m.at[m], sem_m).wait()
        pltpu.make_async_copy(col3.at[s, b], col_m.at[m], sem_m).wait()
        pltpu.make_async_copy(val3.at[s, b], val_m.at[m], sem_m).wait()

    def idx_compute(m):
        for t in range(K // L):
            v = col_m[m, pl.ds(t * L, L)]
            col_m[m, pl.ds(t * L, L)] = v * 2 + c

    def gd(m, buf):
        return pltpu.make_async_copy(x2.at[col_m.at[m]], buf, sem)

    def sd_start(m, lo, hi):
        pltpu.async_copy(lo, acc_lo.at[row_m.at[m]], sem_s, add=True)
        pltpu.async_copy(hi, acc_hi.at[row_m.at[m]], sem_s, add=True)

    def sd_wait(m, lo, hi):
        pltpu.make_async_copy(lo, acc_lo.at[row_m.at[m]], sem_s).wait()
        pltpu.make_async_copy(hi, acc_hi.at[row_m.at[m]], sem_s).wait()

    def scale(m, gbuf, lo, hi):
        def grp_body(g, carry):
            vc = val_m[m, pl.ds(g * L, L)]
            for e in range(L):
                vv = lax.gather(
                    vc, jnp.full((L, 1), e, jnp.int32),
                    lax.GatherDimensionNumbers(
                        offset_dims=(), collapsed_slice_dims=(0,),
                        start_index_map=(0,)),
                    slice_sizes=(1,),
                    mode=lax.GatherScatterMode.PROMISE_IN_BOUNDS)
                r = g * L + e
                for j in range(DH // L):
                    lo[r, pl.ds(j * L, L)] = gbuf[r, pl.ds(j * L, L)] * vv
                for j in range(DH // L):
                    hi[r, pl.ds(j * L, L)] = gbuf[r, pl.ds((4 + j) * L, L)] * vv
            return carry
        lax.fori_loop(0, K // L, grp_body, 0)

    def _zrow(i, carry):
        for j in range(DH // L):
            lo0[i, pl.ds(j * L, L)] = jnp.zeros((L,), jnp.float32)
        return carry
    lax.fori_loop(0, K, _zrow, 0)
    for acc in (acc_lo, acc_hi):
        for t in range(7):
            pltpu.sync_copy(lo0, acc.at[pl.ds(s * ROWS_PER_SUB + t * K, K)])
        pltpu.sync_copy(lo0.at[pl.ds(0, 65)],
                        acc.at[pl.ds(s * ROWS_PER_SUB + 7 * K, 65)])
    plsc.subcore_barrier()

    pltpu.sync_copy(row3.at[s, 0], row_m.at[0])
    pltpu.sync_copy(col3.at[s, 0], col_m.at[0])
    pltpu.sync_copy(val3.at[s, 0], val_m.at[0])
    idx_compute(0)
    gd(0, gbufs[0]).start()
    meta_start(1, 1)

    def emit(b, i, j4=None):
        if j4 is None:
            if b >= 2:
                sd_wait((i + 2) % 4, lobufs[i % 2], hibufs[i % 2])
        elif i >= 2:
            sd_wait((i + 2) % 4, lobufs[i % 2], hibufs[i % 2])
        else:
            @pl.when(j4 >= 1)
            def _():
                sd_wait((i + 2) % 4, lobufs[i % 2], hibufs[i % 2])
        if j4 is not None or b + 1 <= NBATCH - 1:
            meta_wait(b + 1, (i + 1) % 4)
            idx_compute((i + 1) % 4)
            gd((i + 1) % 4, gbufs[(i + 1) % 2]).start()
        if j4 is None:
            if b + 2 <= NBATCH - 1:
                meta_start(b + 2, (i + 2) % 4)
        elif i == GRP - 1:
            @pl.when(j4 <= NGRP - 2)
            def _():
                meta_start(b + 2, (i + 2) % 4)
        else:
            meta_start(b + 2, (i + 2) % 4)
        gd(i % 4, gbufs[i % 2]).wait()
        scale(i % 4, gbufs[i % 2], lobufs[i % 2], hibufs[i % 2])
        sd_start(i % 4, lobufs[i % 2], hibufs[i % 2])

    def body(j4, carry):
        b0 = j4 * GRP
        for i in range(GRP):
            emit(b0 + i, i, j4)
        return carry
    lax.fori_loop(0, NGRP, body, 0)
    for b in range(NGRP * GRP, NBATCH):
        emit(b, b % GRP)
    sd_wait((NBATCH - 2) % 4, lobufs[(NBATCH - 2) % 2], hibufs[(NBATCH - 2) % 2])
    sd_wait((NBATCH - 1) % 4, lobufs[(NBATCH - 1) % 2], hibufs[(NBATCH - 1) % 2])

    plsc.subcore_barrier()

    for acc, q in ((acc_lo, 0), (acc_hi, 1)):
        for t in range(7):
            r0 = s * ROWS_PER_SUB + t * K
            pltpu.sync_copy(acc.at[pl.ds(r0, K)], lo0)
            pltpu.sync_copy(lo0, out_hbm.at[pl.ds(r0, K), 2 * c + q])
        r0 = s * ROWS_PER_SUB + 7 * K
        pltpu.sync_copy(acc.at[pl.ds(r0, 65)], lo0.at[pl.ds(0, 65)])
        pltpu.sync_copy(lo0.at[pl.ds(0, 65)], out_hbm.at[pl.ds(r0, 65), 2 * c + q])


_gcn = pl.kernel(
    _gcn_body,
    out_type=jax.ShapeDtypeStruct((N, NQ, DH), jnp.float32),
    mesh=plsc.VectorSubcoreMesh(core_axis_name="c", subcore_axis_name="s"),
    compiler_params=pltpu.CompilerParams(
        needs_layout_passes=False, use_tc_tiling_on_sc=False),
    scratch_types=[
        pltpu.VMEM((4, K), jnp.int32),
        pltpu.VMEM((4, K), jnp.int32),
        pltpu.VMEM((4, K), jnp.float32),
        pltpu.VMEM((K, DG), jnp.float32),
        pltpu.VMEM((K, DG), jnp.float32),
        pltpu.VMEM((K, DH), jnp.float32),
        pltpu.VMEM((K, DH), jnp.float32),
        pltpu.VMEM((K, DH), jnp.float32),
        pltpu.VMEM((K, DH), jnp.float32),
        pltpu.VMEM_SHARED((N, DH), jnp.float32),
        pltpu.VMEM_SHARED((N, DH), jnp.float32),
        pltpu.SemaphoreType.DMA,
        pltpu.SemaphoreType.DMA,
        pltpu.SemaphoreType.DMA,
    ],
)


def kernel(edge_index, values, x):
    row3 = edge_index[0].reshape(NS, NBATCH, K)
    col3 = edge_index[1].reshape(NS, NBATCH, K)
    val3 = values.reshape(NS, NBATCH, K)
    x2 = x.reshape(2 * N, DG)
    out4 = _gcn(row3, col3, val3, x2)
    return out4.reshape(N, D)


# speedup vs baseline: 1.0975x; 1.0975x over previous
import jax
import jax.numpy as jnp
from jax import lax
from jax.experimental import pallas as pl
from jax.experimental.pallas import tpu as pltpu
from jax.experimental.pallas import tpu_sc as plsc

N = 10000
E = 160000
D = 256
DH = 64
NQ = 4
NC = 2
NS = 16
L = 16

EPS = E // NS
K = 80
NBATCH = EPS // K
ROWS_PER_SUB = N // NS
STRIPE = 125


def _gcn_body(row3, colv, valv, x4, out_hbm,
              row_all, idx_all, val_all, buf0, buf1, buf2, buf3, buf4, buf5,
              stage_v, acc_sh, sem, sem_s):
    c = lax.axis_index("c")
    s = lax.axis_index("s")
    bufs = (buf0, buf1, buf2, buf3, buf4, buf5)

    pltpu.sync_copy(row3.at[s], row_all)
    pltpu.sync_copy(colv.at[s], idx_all)
    pltpu.sync_copy(valv.at[s], val_all)

    def _idx0(i, carry):
        v = idx_all[pl.ds(i * L, L)]
        idx_all[pl.ds(i * L, L)] = v * 4 + c
        return carry
    lax.fori_loop(0, EPS // L, _idx0, 0)

    def gd(b, buf):
        return pltpu.make_async_copy(x4.at[idx_all.at[pl.ds(b * K, K)]], buf, sem)

    def sd_start(b, buf):
        pltpu.async_copy(buf, acc_sh.at[row_all.at[b]], sem_s, add=True)

    def sd_wait(b, buf):
        pltpu.make_async_copy(buf, acc_sh.at[row_all.at[b]], sem_s).wait()

    def scale(b, buf):
        base = b * K

        def grp_body(g, carry):
            vc = val_all[pl.ds(base + g * L, L)]
            for e in range(L):
                vv = lax.gather(
                    vc, jnp.full((L, 1), e, jnp.int32),
                    lax.GatherDimensionNumbers(
                        offset_dims=(), collapsed_slice_dims=(0,),
                        start_index_map=(0,)),
                    slice_sizes=(1,),
                    mode=lax.GatherScatterMode.PROMISE_IN_BOUNDS)
                r = g * L + e
                for j in range(DH // L):
                    buf[r, pl.ds(j * L, L)] = buf[r, pl.ds(j * L, L)] * vv
            return carry
        lax.fori_loop(0, K // L, grp_body, 0)

    for p in range(2):
        if p == 1:
            def _idx1(i, carry):
                idx_all[pl.ds(i * L, L)] = idx_all[pl.ds(i * L, L)] + 2
                return carry
            lax.fori_loop(0, EPS // L, _idx1, 0)

        def _zrow(i, carry):
            for j in range(DH // L):
                stage_v[i, pl.ds(j * L, L)] = jnp.zeros((L,), jnp.float32)
            return carry
        lax.fori_loop(0, STRIPE, _zrow, 0)
        for t in range(ROWS_PER_SUB // STRIPE):
            pltpu.sync_copy(stage_v, acc_sh.at[pl.ds(s * ROWS_PER_SUB + t * STRIPE, STRIPE)])
        plsc.subcore_barrier()

        gd(0, bufs[0]).start()
        gd(1, bufs[1]).start()
        gd(2, bufs[2]).start()

        def emit(b, i, j6=None):
            if j6 is None:
                if b >= 3:
                    sd_wait(b - 3, bufs[(i + 3) % 6])
            elif i >= 3:
                sd_wait(b - 3, bufs[(i + 3) % 6])
            else:
                @pl.when(j6 >= 1)
                def _():
                    sd_wait(b - 3, bufs[(i + 3) % 6])
            if j6 is not None or b + 3 <= NBATCH - 1:
                gd(b + 3, bufs[(i + 3) % 6]).start()
            gd(b, bufs[i % 6]).wait()
            scale(b, bufs[i % 6])
            sd_start(b, bufs[i % 6])

        def body(j6, carry):
            b0 = j6 * 6
            for i in range(6):
                emit(b0 + i, i, j6)
            return carry
        lax.fori_loop(0, 20, body, 0)
        for b in range(120, NBATCH):
            emit(b, b % 6)
        sd_wait(NBATCH - 3, bufs[(NBATCH - 3) % 6])
        sd_wait(NBATCH - 2, bufs[(NBATCH - 2) % 6])
        sd_wait(NBATCH - 1, bufs[(NBATCH - 1) % 6])

        plsc.subcore_barrier()

        for t in range(ROWS_PER_SUB // STRIPE):
            r0 = s * ROWS_PER_SUB + t * STRIPE
            pltpu.sync_copy(acc_sh.at[pl.ds(r0, STRIPE)], stage_v)
            pltpu.sync_copy(stage_v, out_hbm.at[pl.ds(r0, STRIPE), c + 2 * p])


_gcn = pl.kernel(
    _gcn_body,
    out_type=jax.ShapeDtypeStruct((N, NQ, DH), jnp.float32),
    mesh=plsc.VectorSubcoreMesh(core_axis_name="c", subcore_axis_name="s"),
    compiler_params=pltpu.CompilerParams(
        needs_layout_passes=False, use_tc_tiling_on_sc=False),
    scratch_types=[
        pltpu.VMEM((NBATCH, K), jnp.int32),
        pltpu.VMEM((EPS,), jnp.int32),
        pltpu.VMEM((EPS,), jnp.float32),
        pltpu.VMEM((K, DH), jnp.float32),
        pltpu.VMEM((K, DH), jnp.float32),
        pltpu.VMEM((K, DH), jnp.float32),
        pltpu.VMEM((K, DH), jnp.float32),
        pltpu.VMEM((K, DH), jnp.float32),
        pltpu.VMEM((K, DH), jnp.float32),
        pltpu.VMEM((STRIPE, DH), jnp.float32),
        pltpu.VMEM_SHARED((N, DH), jnp.float32),
        pltpu.SemaphoreType.DMA,
        pltpu.SemaphoreType.DMA,
    ],
)


def kernel(edge_index, values, x):
    row3 = edge_index[0].reshape(NS, NBATCH, K)
    colv = edge_index[1].reshape(NS, EPS)
    valv = values.reshape(NS, EPS)
    x4 = x.reshape(NQ * N, DH)
    out4 = _gcn(row3, colv, valv, x4)
    return out4.reshape(N, D)


# speedup vs baseline: 1.9246x; 1.7535x over previous
import jax
import jax.numpy as jnp
from jax import lax
from jax.experimental import pallas as pl
from jax.experimental.pallas import tpu as pltpu
from jax.experimental.pallas import tpu_sc as plsc

N = 10000
E = 160000
D = 256
DH = 64
NQ = 4
NC = 2
NS = 16
L = 16

EPS = E // NS
K = 80
NBATCH = EPS // K
ROWS_PER_SUB = N // NS
STRIPE = 125


def _gcn_body(row3, colv, valv, x4, out_hbm,
              row_all, idx_all, val_all, buf0, buf1, buf2, buf3,
              stage_v, acc_sh, sem, sem_s):
    c = lax.axis_index("c")
    s = lax.axis_index("s")
    bufs = (buf0, buf1, buf2, buf3)

    pltpu.sync_copy(row3.at[s], row_all)
    pltpu.sync_copy(colv.at[s], idx_all)
    pltpu.sync_copy(valv.at[s], val_all)

    def _idx0(i, carry):
        v = idx_all[pl.ds(i * L, L)]
        idx_all[pl.ds(i * L, L)] = v * 4 + c
        return carry
    lax.fori_loop(0, EPS // L, _idx0, 0)

    def gd(b, buf):
        return pltpu.make_async_copy(x4.at[idx_all.at[pl.ds(b * K, K)]], buf, sem)

    def sd_start(b, buf):
        pltpu.async_copy(buf, acc_sh.at[row_all.at[b]], sem_s, add=True)

    def sd_wait(b, buf):
        pltpu.make_async_copy(buf, acc_sh.at[row_all.at[b]], sem_s).wait()

    def scale(b, buf):
        base = b * K
        for g in range(K // L):
            vc = val_all[pl.ds(base + g * L, L)]
            for e in range(L):
                vv = lax.gather(
                    vc, jnp.full((L, 1), e, jnp.int32),
                    lax.GatherDimensionNumbers(
                        offset_dims=(), collapsed_slice_dims=(0,),
                        start_index_map=(0,)),
                    slice_sizes=(1,),
                    mode=lax.GatherScatterMode.PROMISE_IN_BOUNDS)
                r = g * L + e
                for j in range(DH // L):
                    buf[r, pl.ds(j * L, L)] = buf[r, pl.ds(j * L, L)] * vv

    for p in range(2):
        if p == 1:
            def _idx1(i, carry):
                idx_all[pl.ds(i * L, L)] = idx_all[pl.ds(i * L, L)] + 2
                return carry
            lax.fori_loop(0, EPS // L, _idx1, 0)

        def _zrow(i, carry):
            for j in range(DH // L):
                stage_v[i, pl.ds(j * L, L)] = jnp.zeros((L,), jnp.float32)
            return carry
        lax.fori_loop(0, STRIPE, _zrow, 0)
        for t in range(ROWS_PER_SUB // STRIPE):
            pltpu.sync_copy(stage_v, acc_sh.at[pl.ds(s * ROWS_PER_SUB + t * STRIPE, STRIPE)])
        plsc.subcore_barrier()

        gd(0, bufs[0]).start()
        gd(1, bufs[1]).start()

        def body(j4, carry):
            b0 = j4 * 4
            for i in range(4):
                b = b0 + i

                @pl.when(b >= 2)
                def _swait():
                    sd_wait(b - 2, bufs[(i + 2) % 4])

                @pl.when(b + 2 <= NBATCH - 1)
                def _gstart():
                    gd(b + 2, bufs[(i + 2) % 4]).start()

                gd(b, bufs[i]).wait()
                scale(b, bufs[i])
                sd_start(b, bufs[i])
            return carry
        lax.fori_loop(0, NBATCH // 4, body, 0)
        bt = NBATCH - 1
        sd_wait(bt - 2, bufs[2])
        gd(bt, bufs[0]).wait()
        scale(bt, bufs[0])
        sd_start(bt, bufs[0])
        sd_wait(NBATCH - 2, bufs[3])
        sd_wait(NBATCH - 1, bufs[0])

        plsc.subcore_barrier()

        for t in range(ROWS_PER_SUB // STRIPE):
            r0 = s * ROWS_PER_SUB + t * STRIPE
            pltpu.sync_copy(acc_sh.at[pl.ds(r0, STRIPE)], stage_v)
            pltpu.sync_copy(stage_v, out_hbm.at[pl.ds(r0, STRIPE), c + 2 * p])


_gcn = pl.kernel(
    _gcn_body,
    out_type=jax.ShapeDtypeStruct((N, NQ, DH), jnp.float32),
    mesh=plsc.VectorSubcoreMesh(core_axis_name="c", subcore_axis_name="s"),
    compiler_params=pltpu.CompilerParams(
        needs_layout_passes=False, use_tc_tiling_on_sc=False),
    scratch_types=[
        pltpu.VMEM((NBATCH, K), jnp.int32),
        pltpu.VMEM((EPS,), jnp.int32),
        pltpu.VMEM((EPS,), jnp.float32),
        pltpu.VMEM((K, DH), jnp.float32),
        pltpu.VMEM((K, DH), jnp.float32),
        pltpu.VMEM((K, DH), jnp.float32),
        pltpu.VMEM((K, DH), jnp.float32),
        pltpu.VMEM((STRIPE, DH), jnp.float32),
        pltpu.VMEM_SHARED((N, DH), jnp.float32),
        pltpu.SemaphoreType.DMA,
        pltpu.SemaphoreType.DMA,
    ],
)


def kernel(edge_index, values, x):
    row3 = edge_index[0].reshape(NS, NBATCH, K)
    colv = edge_index[1].reshape(NS, EPS)
    valv = values.reshape(NS, EPS)
    x4 = x.reshape(NQ * N, DH)
    out4 = _gcn(row3, colv, valv, x4)
    return out4.reshape(N, D)


# speedup vs baseline: 1.9466x; 1.0114x over previous
import jax
import jax.numpy as jnp
from jax import lax
from jax.experimental import pallas as pl
from jax.experimental.pallas import tpu as pltpu
from jax.experimental.pallas import tpu_sc as plsc

N = 10000
E = 160000
D = 256
DH = 64
NQ = 4
NC = 2
NS = 16
L = 16

EPS = E // NS
K = 80
NBATCH = EPS // K
ROWS_PER_SUB = N // NS
STRIPE = 125


def _gcn_body(row3, colv, valv, x4, out_hbm,
              row_all, idx_all, val_all, buf0, buf1, buf2, buf3,
              stage_v, acc_sh, sem, sem_s):
    c = lax.axis_index("c")
    s = lax.axis_index("s")
    bufs = (buf0, buf1, buf2, buf3)

    pltpu.sync_copy(row3.at[s], row_all)
    pltpu.sync_copy(colv.at[s], idx_all)
    pltpu.sync_copy(valv.at[s], val_all)

    def _idx0(i, carry):
        v = idx_all[pl.ds(i * L, L)]
        idx_all[pl.ds(i * L, L)] = v * 4 + c
        return carry
    lax.fori_loop(0, EPS // L, _idx0, 0)

    def gd(b, buf):
        return pltpu.make_async_copy(x4.at[idx_all.at[pl.ds(b * K, K)]], buf, sem)

    def sd_start(b, buf):
        pltpu.async_copy(buf, acc_sh.at[row_all.at[b]], sem_s, add=True)

    def sd_wait(b, buf):
        pltpu.make_async_copy(buf, acc_sh.at[row_all.at[b]], sem_s).wait()

    def scale(b, buf):
        base = b * K
        for g in range(K // L):
            vc = val_all[pl.ds(base + g * L, L)]
            for e in range(L):
                vv = lax.gather(
                    vc, jnp.full((L, 1), e, jnp.int32),
                    lax.GatherDimensionNumbers(
                        offset_dims=(), collapsed_slice_dims=(0,),
                        start_index_map=(0,)),
                    slice_sizes=(1,),
                    mode=lax.GatherScatterMode.PROMISE_IN_BOUNDS)
                r = g * L + e
                for j in range(DH // L):
                    buf[r, pl.ds(j * L, L)] = buf[r, pl.ds(j * L, L)] * vv

    for p in range(2):
        if p == 1:
            def _idx1(i, carry):
                idx_all[pl.ds(i * L, L)] = idx_all[pl.ds(i * L, L)] + 2
                return carry
            lax.fori_loop(0, EPS // L, _idx1, 0)

        gd(0, bufs[0]).start()
        gd(1, bufs[1]).start()

        def _zrow(i, carry):
            for j in range(DH // L):
                stage_v[i, pl.ds(j * L, L)] = jnp.zeros((L,), jnp.float32)
            return carry
        lax.fori_loop(0, STRIPE, _zrow, 0)
        for t in range(ROWS_PER_SUB // STRIPE):
            pltpu.sync_copy(stage_v, acc_sh.at[pl.ds(s * ROWS_PER_SUB + t * STRIPE, STRIPE)])
        plsc.subcore_barrier()


        def body(j4, carry):
            b0 = j4 * 4
            for i in range(4):
                b = b0 + i

                @pl.when(b >= 2)
                def _swait():
                    sd_wait(b - 2, bufs[(i + 2) % 4])

                @pl.when(b + 2 <= NBATCH - 1)
                def _gstart():
                    gd(b + 2, bufs[(i + 2) % 4]).start()

                gd(b, bufs[i]).wait()
                scale(b, bufs[i])
                sd_start(b, bufs[i])
            return carry
        lax.fori_loop(0, NBATCH // 4, body, 0)
        bt = NBATCH - 1
        sd_wait(bt - 2, bufs[2])
        gd(bt, bufs[0]).wait()
        scale(bt, bufs[0])
        sd_start(bt, bufs[0])
        sd_wait(NBATCH - 2, bufs[3])
        sd_wait(NBATCH - 1, bufs[0])

        plsc.subcore_barrier()

        for t in range(ROWS_PER_SUB // STRIPE):
            r0 = s * ROWS_PER_SUB + t * STRIPE
            pltpu.sync_copy(acc_sh.at[pl.ds(r0, STRIPE)], stage_v)
            pltpu.sync_copy(stage_v, out_hbm.at[pl.ds(r0, STRIPE), c + 2 * p])


_gcn = pl.kernel(
    _gcn_body,
    out_type=jax.ShapeDtypeStruct((N, NQ, DH), jnp.float32),
    mesh=plsc.VectorSubcoreMesh(core_axis_name="c", subcore_axis_name="s"),
    compiler_params=pltpu.CompilerParams(
        needs_layout_passes=False, use_tc_tiling_on_sc=False),
    scratch_types=[
        pltpu.VMEM((NBATCH, K), jnp.int32),
        pltpu.VMEM((EPS,), jnp.int32),
        pltpu.VMEM((EPS,), jnp.float32),
        pltpu.VMEM((K, DH), jnp.float32),
        pltpu.VMEM((K, DH), jnp.float32),
        pltpu.VMEM((K, DH), jnp.float32),
        pltpu.VMEM((K, DH), jnp.float32),
        pltpu.VMEM((STRIPE, DH), jnp.float32),
        pltpu.VMEM_SHARED((N, DH), jnp.float32),
        pltpu.SemaphoreType.DMA,
        pltpu.SemaphoreType.DMA,
    ],
)


def kernel(edge_index, values, x):
    row3 = edge_index[0].reshape(NS, NBATCH, K)
    colv = edge_index[1].reshape(NS, EPS)
    valv = values.reshape(NS, EPS)
    x4 = x.reshape(NQ * N, DH)
    out4 = _gcn(row3, colv, valv, x4)
    return out4.reshape(N, D)
